# baseline (device time: 64741 ns/iter reference)
import jax
import jax.numpy as jnp
from jax import lax
from jax.experimental import pallas as pl
from jax.experimental.pallas import tpu as pltpu

N_DEV = 16
B, SQ, D = 4, 256, 1024
H_LOC, DH = 8, 128
ROWS = B * SQ
CHUNK = ROWS // N_DEV
SCALE = 0.08838834764831843


def kernel(x, Wq, Wo, Wk, Wv):
    x2 = x.reshape(ROWS, D)

    def body(x_ref, wq_ref, wk_ref, wv_ref, wo_ref, out_ref,
             attn_ref, stageA_ref, slotA_ref, stageB_ref, slotB_ref,
             sendA, recvA, sendB, recvB, sendC, recvC, sendD, recvD):
        d = lax.axis_index("i")
        w = lax.rem(d, 4)
        z = lax.div(d, 4)

        QR, SC = 256, 64
        pending = []

        wq_v = (wq_ref[:] * SCALE).astype(jnp.bfloat16)
        wk_v = wk_ref[:].astype(jnp.bfloat16)
        wv_v = wv_ref[:].astype(jnp.bfloat16)
        wo_v = wo_ref[:].astype(jnp.bfloat16)

        for j in (1, 2, 3, 0):
            b = lax.rem(w + j, 4)
            r0 = b * QR
            xb_b = x_ref[pl.ds(r0, QR), :].astype(jnp.bfloat16)
            qb = jnp.dot(xb_b, wq_v,
                         preferred_element_type=jnp.float32).astype(
                             jnp.bfloat16)
            kb = jnp.dot(xb_b, wk_v,
                         preferred_element_type=jnp.float32).astype(
                             jnp.bfloat16)
            vb = jnp.dot(xb_b, wv_v,
                         preferred_element_type=jnp.float32).astype(
                             jnp.bfloat16)
            for h in range(H_LOC):
                qs = qb[:, h * DH:(h + 1) * DH]
                ks = kb[:, h * DH:(h + 1) * DH]
                vs = vb[:, h * DH:(h + 1) * DH]
                s = lax.dot_general(
                    qs, ks, (((1,), (1,)), ((), ())),
                    preferred_element_type=jnp.float32,
                )
                m = jnp.max(s, axis=1, keepdims=True)
                p = jnp.exp(s - m)
                l = jnp.sum(p, axis=1, keepdims=True)
                o = jnp.dot(p.astype(jnp.bfloat16), vs,
                            preferred_element_type=jnp.float32) * (1.0 / l)
                attn_ref[:, h * DH:(h + 1) * DH] = o.astype(jnp.bfloat16)
            pb = jnp.dot(attn_ref[:], wo_v,
                         preferred_element_type=jnp.float32).astype(
                             jnp.bfloat16)
            if j == 0:
                slotA_ref[pl.ds(w * QR, QR), :] = pb
            else:
                stageA_ref[pl.ds(r0, QR), :] = pb
                peer = z * 4 + b
                rdma = pltpu.make_async_remote_copy(
                    src_ref=stageA_ref.at[pl.ds(r0, QR), :],
                    dst_ref=slotA_ref.at[pl.ds(w * QR, QR), :],
                    send_sem=sendA.at[j],
                    recv_sem=recvA.at[j],
                    device_id=(peer,),
                    device_id_type=pl.DeviceIdType.MESH,
                )
                rdma.start()
                pending.append(rdma)

        for j in range(1, 4):
            ws = lax.rem(w - j + 4, 4)
            recv = pltpu.make_async_remote_copy(
                src_ref=stageA_ref.at[pl.ds(0, QR), :],
                dst_ref=slotA_ref.at[pl.ds(ws * QR, QR), :],
                send_sem=sendA.at[j],
                recv_sem=recvA.at[j],
                device_id=(d,),
                device_id_type=pl.DeviceIdType.MESH,
            )
            recv.wait_recv()
        qsum = (slotA_ref[pl.ds(0 * QR, QR), :].astype(jnp.float32)
                + slotA_ref[pl.ds(1 * QR, QR), :].astype(jnp.float32)
                + slotA_ref[pl.ds(2 * QR, QR), :].astype(jnp.float32)
                + slotA_ref[pl.ds(3 * QR, QR), :].astype(jnp.float32))

        stageB_ref[:] = qsum.astype(jnp.bfloat16)
        for j in range(1, 4):
            zp = lax.rem(z + j, 4)
            peer = zp * 4 + w
            rdma = pltpu.make_async_remote_copy(
                src_ref=stageB_ref.at[pl.ds(zp * SC, SC), :],
                dst_ref=slotB_ref.at[pl.ds(z * SC, SC), :],
                send_sem=sendB.at[j],
                recv_sem=recvB.at[j],
                device_id=(peer,),
                device_id_type=pl.DeviceIdType.MESH,
            )
            rdma.start()
            pending.append(rdma)
        slotB_ref[pl.ds(z * SC, SC), :] = stageB_ref[pl.ds(z * SC, SC), :]
        for j in range(1, 4):
            zs = lax.rem(z - j + 4, 4)
            recv = pltpu.make_async_remote_copy(
                src_ref=stageB_ref.at[pl.ds(0, SC), :],
                dst_ref=slotB_ref.at[pl.ds(zs * SC, SC), :],
                send_sem=sendB.at[j],
                recv_sem=recvB.at[j],
                device_id=(d,),
                device_id_type=pl.DeviceIdType.MESH,
            )
            recv.wait_recv()
        final = (slotB_ref[pl.ds(0 * SC, SC), :].astype(jnp.float32)
                 + slotB_ref[pl.ds(1 * SC, SC), :].astype(jnp.float32)
                 + slotB_ref[pl.ds(2 * SC, SC), :].astype(jnp.float32)
                 + slotB_ref[pl.ds(3 * SC, SC), :].astype(jnp.float32))
        my_rows = w * QR + z * SC
        out_ref[pl.ds(my_rows, SC), :] = final.astype(jnp.bfloat16)

        for j in range(1, 4):
            zp = lax.rem(z + j, 4)
            peer = zp * 4 + w
            rdma = pltpu.make_async_remote_copy(
                src_ref=out_ref.at[pl.ds(my_rows, SC), :],
                dst_ref=out_ref.at[pl.ds(my_rows, SC), :],
                send_sem=sendC.at[j],
                recv_sem=recvC.at[j],
                device_id=(peer,),
                device_id_type=pl.DeviceIdType.MESH,
            )
            rdma.start()
            pending.append(rdma)

        for k in range(4):
            zs = lax.rem(z - k + 4, 4)
            rows_k = w * QR + zs * SC
            if k > 0:
                recv = pltpu.make_async_remote_copy(
                    src_ref=out_ref.at[pl.ds(0, SC), :],
                    dst_ref=out_ref.at[pl.ds(rows_k, SC), :],
                    send_sem=sendC.at[k],
                    recv_sem=recvC.at[k],
                    device_id=(d,),
                    device_id_type=pl.DeviceIdType.MESH,
                )
                recv.wait_recv()
            for j in range(1, 4):
                wp = lax.rem(w + j, 4)
                peer = z * 4 + wp
                rdma = pltpu.make_async_remote_copy(
                    src_ref=out_ref.at[pl.ds(rows_k, SC), :],
                    dst_ref=out_ref.at[pl.ds(rows_k, SC), :],
                    send_sem=sendD.at[k * 4 + j],
                    recv_sem=recvD.at[k * 4 + j],
                    device_id=(peer,),
                    device_id_type=pl.DeviceIdType.MESH,
                )
                rdma.start()
                pending.append(rdma)

        for k in range(4):
            zs = lax.rem(z - k + 4, 4)
            for j in range(1, 4):
                ws = lax.rem(w - j + 4, 4)
                recv = pltpu.make_async_remote_copy(
                    src_ref=out_ref.at[pl.ds(0, SC), :],
                    dst_ref=out_ref.at[pl.ds(ws * QR + zs * SC, SC), :],
                    send_sem=sendD.at[k * 4 + j],
                    recv_sem=recvD.at[k * 4 + j],
                    device_id=(d,),
                    device_id_type=pl.DeviceIdType.MESH,
                )
                recv.wait_recv()

        for rdma in pending:
            rdma.wait_send()

    out2 = pl.pallas_call(
        body,
        out_shape=jax.ShapeDtypeStruct((ROWS, D), jnp.bfloat16),
        in_specs=[pl.BlockSpec(memory_space=pltpu.VMEM)] * 5,
        out_specs=pl.BlockSpec(memory_space=pltpu.VMEM),
        scratch_shapes=[
            pltpu.VMEM((256, D), jnp.bfloat16),
            pltpu.VMEM((ROWS, D), jnp.bfloat16),
            pltpu.VMEM((ROWS, D), jnp.bfloat16),
            pltpu.VMEM((256, D), jnp.bfloat16),
            pltpu.VMEM((256, D), jnp.bfloat16),
            pltpu.SemaphoreType.DMA((4,)),
            pltpu.SemaphoreType.DMA((4,)),
            pltpu.SemaphoreType.DMA((4,)),
            pltpu.SemaphoreType.DMA((4,)),
            pltpu.SemaphoreType.DMA((4,)),
            pltpu.SemaphoreType.DMA((4,)),
            pltpu.SemaphoreType.DMA((16,)),
            pltpu.SemaphoreType.DMA((16,)),
        ],
    )(x2, Wq, Wk, Wv, Wo)
    return out2.reshape(B, SQ, D)
